# baseline (speedup 1.0000x reference)
import functools

import jax
import jax.numpy as jnp
from jax import lax
from jax.experimental import pallas as pl
from jax.experimental.pallas import tpu as pltpu

EPS = 1e-5
LEAK = 0.01


def _rup(a, b):
    return (a + b - 1) // b * b


def _lstm_body(md_ref, counts_ref, wih_ref, whh_ref, bias_ref, tmx_ref,
               h_ref, c_ref, xbuf_ref, sems, *, ch, nch):
    h_ref[...] = jnp.zeros_like(h_ref)
    c_ref[...] = jnp.zeros_like(c_ref)
    max_deg = md_ref[0]
    counts = counts_ref[...]
    hdim = whh_ref.shape[0]

    def step(carry):
        t, off = carry
        n_t = jnp.sum((counts > t).astype(jnp.int32))
        nact = (n_t + ch - 1) // ch

        def fire(ci, _):
            pltpu.make_async_copy(
                tmx_ref.at[pl.ds(off + ci * ch, ch)],
                xbuf_ref.at[ci], sems.at[ci]).start()
            return 0
        lax.fori_loop(0, nact, fire, 0, unroll=False)

        def common(ci, with_x):
            row0 = pl.multiple_of(ci * ch, ch)
            hc = h_ref[pl.ds(row0, ch), :]
            cc = c_ref[pl.ds(row0, ch), :]
            gates = jnp.dot(hc.astype(jnp.bfloat16), whh_ref[...],
                            preferred_element_type=jnp.float32) + bias_ref[...]
            if with_x:
                pltpu.make_async_copy(
                    tmx_ref.at[pl.ds(off + ci * ch, ch)],
                    xbuf_ref.at[ci], sems.at[ci]).wait()
                nrows = n_t - row0
                rmask = lax.broadcasted_iota(
                    jnp.int32, xbuf_ref.shape[1:], 0) < nrows
                xt = jnp.where(rmask, xbuf_ref[ci], 0.0)
                gates = gates + jnp.dot(xt.astype(jnp.bfloat16), wih_ref[...],
                                        preferred_element_type=jnp.float32)
            i = jax.nn.sigmoid(gates[:, 0:hdim])
            f = jax.nn.sigmoid(gates[:, hdim:2 * hdim])
            g = jnp.tanh(gates[:, 2 * hdim:3 * hdim])
            o = jax.nn.sigmoid(gates[:, 3 * hdim:4 * hdim])
            cc = f * cc + i * g
            h_ref[pl.ds(row0, ch), :] = o * jnp.tanh(cc)
            c_ref[pl.ds(row0, ch), :] = cc
            return 0

        lax.fori_loop(0, nact, lambda ci, _: common(ci, True), 0,
                      unroll=False)
        lax.fori_loop(nact, nch, lambda ci, _: common(ci, False), 0,
                      unroll=False)
        return t + 1, off + n_t

    lax.while_loop(lambda carry: carry[0] < max_deg, step,
                   (jnp.int32(0), jnp.int32(0)))


def _run_lstm(max_deg, counts_sorted_pad, wih_t, whh_t, bias, tmx, *,
              np_, ch, d, h):
    nch = np_ // ch
    body = functools.partial(_lstm_body, ch=ch, nch=nch)
    return pl.pallas_call(
        body,
        grid=(),
        in_specs=[
            pl.BlockSpec(memory_space=pltpu.MemorySpace.SMEM),
            pl.BlockSpec(memory_space=pltpu.MemorySpace.VMEM),
            pl.BlockSpec(memory_space=pltpu.MemorySpace.VMEM),
            pl.BlockSpec(memory_space=pltpu.MemorySpace.VMEM),
            pl.BlockSpec(memory_space=pltpu.MemorySpace.VMEM),
            pl.BlockSpec(memory_space=pltpu.MemorySpace.HBM),
        ],
        out_specs=pl.BlockSpec(memory_space=pltpu.MemorySpace.VMEM),
        out_shape=jax.ShapeDtypeStruct((np_, h), jnp.float32),
        scratch_shapes=[
            pltpu.VMEM((np_, h), jnp.float32),
            pltpu.VMEM((nch, ch, d), jnp.float32),
            pltpu.SemaphoreType.DMA((nch,)),
        ],
    )(max_deg, counts_sorted_pad, wih_t, whh_t, bias, tmx)


def _post_body(aggr_ref, x_ref, batch_ref, wl_ref, wr_ref, bl_ref,
               g1_ref, b1_ref, wd_ref, bd_ref, g2_ref, b2_ref,
               wreg_ref, breg_ref, out_ref, *, n, g):
    hv = (jnp.dot(aggr_ref[...].astype(jnp.bfloat16), wl_ref[...],
                  preferred_element_type=jnp.float32) + bl_ref[...]
          + jnp.dot(x_ref[...].astype(jnp.bfloat16), wr_ref[...],
                    preferred_element_type=jnp.float32))
    mean = jnp.mean(hv, axis=0, keepdims=True)
    var = jnp.mean((hv - mean) ** 2, axis=0, keepdims=True)
    hv = g1_ref[...] * (hv - mean) * jax.lax.rsqrt(var + EPS) + b1_ref[...]
    hv = jnp.where(hv > 0, hv, LEAK * hv)
    seg = lax.broadcasted_iota(jnp.int32, (g, n), 0)
    onehot_t = (seg == batch_ref[...]).astype(jnp.float32)
    sums = jnp.dot(onehot_t, hv, preferred_element_type=jnp.float32,
                   precision=lax.Precision.HIGHEST)
    cnt = jnp.sum(onehot_t, axis=1, keepdims=True)
    pooled = sums / jnp.maximum(cnt, 1.0)
    z = jnp.dot(pooled.astype(jnp.bfloat16), wd_ref[...],
                preferred_element_type=jnp.float32) + bd_ref[...]
    mean2 = jnp.mean(z, axis=0, keepdims=True)
    var2 = jnp.mean((z - mean2) ** 2, axis=0, keepdims=True)
    z = g2_ref[...] * (z - mean2) * jax.lax.rsqrt(var2 + EPS) + b2_ref[...]
    z = jnp.where(z > 0, z, LEAK * z)
    out_ref[...] = (jnp.dot(z.astype(jnp.bfloat16), wreg_ref[...],
                            preferred_element_type=jnp.float32)
                    + breg_ref[...])


def _run_post(aggr, x, batch_row, wl_t, wr_t, bl, g1, b1, wd_t, bd,
              g2, b2, wreg_t, breg, *, n, g):
    body = functools.partial(_post_body, n=n, g=g)
    vmem = pl.BlockSpec(memory_space=pltpu.MemorySpace.VMEM)
    return pl.pallas_call(
        body,
        grid=(),
        in_specs=[vmem] * 14,
        out_specs=vmem,
        out_shape=jax.ShapeDtypeStruct((g, 1), jnp.float32),
    )(aggr, x, batch_row, wl_t, wr_t, bl, g1, b1, wd_t, bd, g2, b2,
      wreg_t, breg)


def kernel(x, edge_index, batch, w_ih, w_hh, b_ih, b_hh, w_l, b_l, w_r,
           bn1_g, bn1_b, w_d, b_d, bn2_g, bn2_b, w_reg, b_reg):
    n, d = x.shape
    e = edge_index.shape[1]
    h = w_hh.shape[1]
    g = 64

    ch = min(512, _rup(n, 8))
    np_ = _rup(n, ch)
    b_tm = _rup(e + ch, 4096)

    src = edge_index[0]
    dst = edge_index[1]
    key = dst.astype(jnp.int32) * n + src.astype(jnp.int32)
    perm = jnp.argsort(key)
    src_s = src[perm]
    dst_s = dst[perm]

    counts = jnp.bincount(dst, length=n).astype(jnp.int32)
    starts = jnp.concatenate(
        [jnp.zeros((1,), jnp.int32),
         jnp.cumsum(counts)[:-1].astype(jnp.int32)])
    max_deg = jnp.max(counts)

    order = jnp.argsort(-counts)
    counts_sorted = counts[order]
    rank = jnp.zeros((n,), jnp.int32).at[order].set(
        jnp.arange(n, dtype=jnp.int32))

    hist = jnp.bincount(counts, length=e + 1)
    cnt_le = jnp.cumsum(hist)
    n_of_t = (n - cnt_le[:e]).astype(jnp.int32)
    off = jnp.concatenate(
        [jnp.zeros((1,), jnp.int32), jnp.cumsum(n_of_t).astype(jnp.int32)])

    k_arr = jnp.arange(e, dtype=jnp.int32) - starts[dst_s]
    p_arr = off[k_arr] + rank[dst_s]
    tm_src = jnp.zeros((b_tm,), jnp.int32).at[p_arr].set(src_s)

    tmx = jnp.take(x, tm_src, axis=0)

    counts_pad = jnp.zeros((np_,), jnp.int32).at[:n].set(counts_sorted)
    counts_2d = counts_pad.reshape(1, np_)

    bias = (b_ih + b_hh).reshape(1, 4 * h)
    h_ranked = _run_lstm(
        max_deg.reshape(1).astype(jnp.int32), counts_2d,
        w_ih.T.astype(jnp.bfloat16), w_hh.T.astype(jnp.bfloat16),
        bias, tmx, np_=np_, ch=ch, d=d, h=h)

    aggr = jnp.take(h_ranked, rank, axis=0)

    out = _run_post(
        aggr, x, batch.reshape(1, n).astype(jnp.int32),
        w_l.T.astype(jnp.bfloat16), w_r.T.astype(jnp.bfloat16),
        b_l.reshape(1, h), bn1_g.reshape(1, h),
        bn1_b.reshape(1, h), w_d.T.astype(jnp.bfloat16), b_d.reshape(1, -1),
        bn2_g.reshape(1, -1), bn2_b.reshape(1, -1),
        w_reg.T.astype(jnp.bfloat16),
        b_reg.reshape(1, 1), n=n, g=g)
    return out


# speedup vs baseline: 2.7766x; 2.7766x over previous
import functools

import jax
import jax.numpy as jnp
from jax import lax
from jax.experimental import pallas as pl
from jax.experimental.pallas import tpu as pltpu

EPS = 1e-5
LEAK = 0.01


def _rup(a, b):
    return (a + b - 1) // b * b


def _lstm_body(md_ref, counts_ref, wih_ref, whh_ref, bias_ref, tmx_ref,
               h_ref, c_ref, xbuf_ref, sems, *, ch, nch):
    h_ref[...] = jnp.zeros_like(h_ref)
    c_ref[...] = jnp.zeros_like(c_ref)
    max_deg = md_ref[0]
    counts = counts_ref[...]
    hdim = whh_ref.shape[0]

    def step(carry):
        t, off = carry
        n_t = jnp.sum((counts > t).astype(jnp.int32))
        nact = (n_t + ch - 1) // ch

        def fire(ci, _):
            pltpu.make_async_copy(
                tmx_ref.at[pl.ds(off + ci * ch, ch)],
                xbuf_ref.at[ci], sems.at[ci]).start()
            return 0
        lax.fori_loop(0, nact, fire, 0, unroll=False)

        def common(ci, with_x):
            row0 = pl.multiple_of(ci * ch, ch)
            hc = h_ref[pl.ds(row0, ch), :]
            cc = c_ref[pl.ds(row0, ch), :]
            gates = jnp.dot(hc.astype(jnp.bfloat16), whh_ref[...],
                            preferred_element_type=jnp.float32) + bias_ref[...]
            if with_x:
                pltpu.make_async_copy(
                    tmx_ref.at[pl.ds(off + ci * ch, ch)],
                    xbuf_ref.at[ci], sems.at[ci]).wait()
                nrows = n_t - row0
                rmask = lax.broadcasted_iota(
                    jnp.int32, xbuf_ref.shape[1:], 0) < nrows
                xt = jnp.where(rmask, xbuf_ref[ci], 0.0)
                gates = gates + jnp.dot(xt.astype(jnp.bfloat16), wih_ref[...],
                                        preferred_element_type=jnp.float32)
            i = jax.nn.sigmoid(gates[:, 0:hdim])
            f = jax.nn.sigmoid(gates[:, hdim:2 * hdim])
            g = jnp.tanh(gates[:, 2 * hdim:3 * hdim])
            o = jax.nn.sigmoid(gates[:, 3 * hdim:4 * hdim])
            cc = f * cc + i * g
            h_ref[pl.ds(row0, ch), :] = o * jnp.tanh(cc)
            c_ref[pl.ds(row0, ch), :] = cc
            return 0

        lax.fori_loop(0, nact, lambda ci, _: common(ci, True), 0,
                      unroll=False)
        lax.fori_loop(nact, nch, lambda ci, _: common(ci, False), 0,
                      unroll=False)
        return t + 1, off + n_t

    lax.while_loop(lambda carry: carry[0] < max_deg, step,
                   (jnp.int32(0), jnp.int32(0)))


def _run_lstm(max_deg, counts_sorted_pad, wih_t, whh_t, bias, tmx, *,
              np_, ch, d, h):
    nch = np_ // ch
    body = functools.partial(_lstm_body, ch=ch, nch=nch)
    return pl.pallas_call(
        body,
        grid=(),
        in_specs=[
            pl.BlockSpec(memory_space=pltpu.MemorySpace.SMEM),
            pl.BlockSpec(memory_space=pltpu.MemorySpace.VMEM),
            pl.BlockSpec(memory_space=pltpu.MemorySpace.VMEM),
            pl.BlockSpec(memory_space=pltpu.MemorySpace.VMEM),
            pl.BlockSpec(memory_space=pltpu.MemorySpace.VMEM),
            pl.BlockSpec(memory_space=pltpu.MemorySpace.HBM),
        ],
        out_specs=pl.BlockSpec(memory_space=pltpu.MemorySpace.VMEM),
        out_shape=jax.ShapeDtypeStruct((np_, h), jnp.float32),
        scratch_shapes=[
            pltpu.VMEM((np_, h), jnp.float32),
            pltpu.VMEM((nch, ch, d), jnp.float32),
            pltpu.SemaphoreType.DMA((nch,)),
        ],
    )(max_deg, counts_sorted_pad, wih_t, whh_t, bias, tmx)


def _post_body(aggr_ref, x_ref, batch_ref, wl_ref, wr_ref, bl_ref,
               g1_ref, b1_ref, wd_ref, bd_ref, g2_ref, b2_ref,
               wreg_ref, breg_ref, out_ref, *, n, g):
    hv = (jnp.dot(aggr_ref[...].astype(jnp.bfloat16), wl_ref[...],
                  preferred_element_type=jnp.float32) + bl_ref[...]
          + jnp.dot(x_ref[...].astype(jnp.bfloat16), wr_ref[...],
                    preferred_element_type=jnp.float32))
    mean = jnp.mean(hv, axis=0, keepdims=True)
    var = jnp.mean((hv - mean) ** 2, axis=0, keepdims=True)
    hv = g1_ref[...] * (hv - mean) * jax.lax.rsqrt(var + EPS) + b1_ref[...]
    hv = jnp.where(hv > 0, hv, LEAK * hv)
    seg = lax.broadcasted_iota(jnp.int32, (g, n), 0)
    onehot_t = (seg == batch_ref[...]).astype(jnp.float32)
    sums = jnp.dot(onehot_t, hv, preferred_element_type=jnp.float32,
                   precision=lax.Precision.HIGHEST)
    cnt = jnp.sum(onehot_t, axis=1, keepdims=True)
    pooled = sums / jnp.maximum(cnt, 1.0)
    z = jnp.dot(pooled.astype(jnp.bfloat16), wd_ref[...],
                preferred_element_type=jnp.float32) + bd_ref[...]
    mean2 = jnp.mean(z, axis=0, keepdims=True)
    var2 = jnp.mean((z - mean2) ** 2, axis=0, keepdims=True)
    z = g2_ref[...] * (z - mean2) * jax.lax.rsqrt(var2 + EPS) + b2_ref[...]
    z = jnp.where(z > 0, z, LEAK * z)
    out_ref[...] = (jnp.dot(z.astype(jnp.bfloat16), wreg_ref[...],
                            preferred_element_type=jnp.float32)
                    + breg_ref[...])


def _run_post(aggr, x, batch_row, wl_t, wr_t, bl, g1, b1, wd_t, bd,
              g2, b2, wreg_t, breg, *, n, g):
    body = functools.partial(_post_body, n=n, g=g)
    vmem = pl.BlockSpec(memory_space=pltpu.MemorySpace.VMEM)
    return pl.pallas_call(
        body,
        grid=(),
        in_specs=[vmem] * 14,
        out_specs=vmem,
        out_shape=jax.ShapeDtypeStruct((g, 1), jnp.float32),
    )(aggr, x, batch_row, wl_t, wr_t, bl, g1, b1, wd_t, bd, g2, b2,
      wreg_t, breg)


def kernel(x, edge_index, batch, w_ih, w_hh, b_ih, b_hh, w_l, b_l, w_r,
           bn1_g, bn1_b, w_d, b_d, bn2_g, bn2_b, w_reg, b_reg):
    n, d = x.shape
    e = edge_index.shape[1]
    h = w_hh.shape[1]
    g = 64

    ch = min(512, _rup(n, 8))
    np_ = _rup(n, ch)
    b_tm = _rup(e + ch, 4096)

    src = edge_index[0].astype(jnp.int32)
    dst = edge_index[1].astype(jnp.int32)

    counts = jnp.bincount(dst, length=n).astype(jnp.int32)
    max_deg = jnp.max(counts)

    _, order, counts_sorted = lax.sort(
        [-counts, jnp.arange(n, dtype=jnp.int32), counts], num_keys=1)
    rank = jnp.zeros((n,), jnp.int32).at[order].set(
        jnp.arange(n, dtype=jnp.int32))

    m_tot = n + e
    key_m = jnp.concatenate([
        (jnp.arange(n, dtype=jnp.int32) * 2) * n,
        (dst * 2 + 1) * n + src,
    ])
    val_m = jnp.concatenate([rank, src])
    keym_s, valm_s = lax.sort([key_m, val_m], num_keys=1)
    idq = keym_s // n
    is_node = (idq & 1) == 0
    pos = jnp.arange(m_tot, dtype=jnp.int32)
    r_hi = valm_s >> 7
    r_lo = valm_s & 127
    p_hi = lax.cummax(jnp.where(is_node, pos * 128 + r_hi, -1))
    p_lo = lax.cummax(jnp.where(is_node, pos * 128 + r_lo, -1))
    rank_b = ((p_hi & 127) << 7) | (p_lo & 127)
    nodepos_b = lax.cummax(jnp.where(is_node, pos, -1))
    k_arr = pos - nodepos_b - 1

    key3 = jnp.where(is_node, jnp.int32(2**31 - 1), k_arr * np_ + rank_b)
    _, tm_src_full = lax.sort([key3, valm_s], num_keys=1)
    tm_src = jnp.concatenate(
        [tm_src_full[:e], jnp.zeros((b_tm - e,), jnp.int32)])

    tmx = jnp.take(x, tm_src, axis=0)

    counts_pad = jnp.zeros((np_,), jnp.int32).at[:n].set(counts_sorted)
    counts_2d = counts_pad.reshape(1, np_)

    bias = (b_ih + b_hh).reshape(1, 4 * h)
    h_ranked = _run_lstm(
        max_deg.reshape(1).astype(jnp.int32), counts_2d,
        w_ih.T.astype(jnp.bfloat16), w_hh.T.astype(jnp.bfloat16),
        bias, tmx, np_=np_, ch=ch, d=d, h=h)

    aggr = jnp.take(h_ranked, rank, axis=0)

    out = _run_post(
        aggr, x, batch.reshape(1, n).astype(jnp.int32),
        w_l.T.astype(jnp.bfloat16), w_r.T.astype(jnp.bfloat16),
        b_l.reshape(1, h), bn1_g.reshape(1, h),
        bn1_b.reshape(1, h), w_d.T.astype(jnp.bfloat16), b_d.reshape(1, -1),
        bn2_g.reshape(1, -1), bn2_b.reshape(1, -1),
        w_reg.T.astype(jnp.bfloat16),
        b_reg.reshape(1, 1), n=n, g=g)
    return out
